# baseline (speedup 1.0000x reference)
import jax
import jax.numpy as jnp
import numpy as np
from jax import lax
from jax.experimental import pallas as pl

NSUB = 16
BATCH_BLOCK = 512
GROUP_BLOCK = 8
NCLUST = 4


def _up(arr, s):
    return jnp.concatenate([arr[s:], arr[:s]], axis=0)


def _down(arr, s):
    n = arr.shape[0]
    return jnp.concatenate([arr[n - s:], arr[:n - s]], axis=0)


def _hoa_body(x_ref, c_ref, o_ref):
    nd = GROUP_BLOCK * 32
    nr = NCLUST * GROUP_BLOCK * 4
    xb = x_ref[...]
    xt = jnp.transpose(xb.reshape(xb.shape[0], -1))
    r1, r2, r3 = _up(xt, 1), _up(xt, 2), _up(xt, 3)
    p1 = jnp.minimum(xt, r1)
    p2 = jnp.minimum(xt, r2)
    p3 = jnp.minimum(xt, r3)
    t1 = jnp.minimum(p1, r2)
    t2 = jnp.minimum(p1, r3)
    t3 = jnp.minimum(p2, r3)
    q = jnp.minimum(t1, r3)
    mx = jnp.maximum(jnp.maximum(xt, r1), jnp.maximum(r2, r3))
    pos = jax.lax.broadcasted_iota(jnp.int32, xt.shape, 0) % 4
    A0 = xt
    A1 = jnp.where(pos == 3, _down(t2, 3), p1)
    A2 = jnp.where(pos < 2, p2, _down(t1, 2))
    A3 = jnp.where(pos == 0, p3,
                   jnp.where(pos == 1, _down(t3, 1),
                             jnp.where(pos == 2, _down(q, 2), _down(mx, 3))))
    for jj in range(NCLUST):
        s = jj * 4 * GROUP_BLOCK
        e = s + 4 * GROUP_BLOCK
        m = jnp.concatenate([A0[s:e], A1[s:e], A2[s:e], A3[s:e]], axis=0)
        o_ref[:, jj * nd:(jj + 1) * nd] = lax.dot_general(
            m, c_ref[jj], (((0,), (0,)), ((), ())),
            preferred_element_type=jnp.float32,
            precision=lax.Precision.HIGHEST)


def kernel(X, params):
    B, G, A = X.shape
    D = params.shape[-1]
    GB, BB = GROUP_BLOCK, BATCH_BLOCK


    c = params.reshape(G, 2, 2, 2, 2, D)
    for ax in (1, 2, 3, 4):
        lo = lax.slice_in_dim(c, 0, 1, axis=ax)
        hi = lax.slice_in_dim(c, 1, 2, axis=ax)
        c = jnp.concatenate([lo, hi - lo], axis=ax)
    cmob = c.reshape(G, NSUB, D)
    cmob = cmob.at[:, 0, :].set(params[:, 0, :])
    tidx = np.array([[1, 2, 4, 8],
                     [3, 6, 12, 11],
                     [5, 10, 7, 14],
                     [9, 13, 15, 0]], dtype=np.int32)
    cm = cmob.reshape(G // GB, GB, NSUB, D)
    sel = jnp.take(cm, jnp.asarray(tidx.reshape(-1)), axis=2)
    sel = sel.reshape(G // GB, GB, 4, 4, D)
    cbd = jnp.einsum('cjkid,jJ->ckjiJd', sel, jnp.eye(GB, dtype=cmob.dtype))
    cbd = cbd.reshape(G // GB, NSUB * GB, GB * D)

    grid = (G // (GB * NCLUST), B // BB)
    out2 = pl.pallas_call(
        _hoa_body,
        grid=grid,
        in_specs=[
            pl.BlockSpec((BB, NCLUST * GB, A), lambda g, b: (b, g, 0)),
            pl.BlockSpec((NCLUST, NSUB * GB, GB * D), lambda g, b: (g, 0, 0)),
        ],
        out_specs=pl.BlockSpec((BB, NCLUST * GB * D), lambda g, b: (b, g)),
        out_shape=jax.ShapeDtypeStruct((B, G * D), jnp.float32),
    )(X, cbd)
    return out2.reshape(B, G, D)


# speedup vs baseline: 1.9675x; 1.9675x over previous
import jax
import jax.numpy as jnp
import numpy as np
from jax import lax
from jax.experimental import pallas as pl

NSUB = 16
BATCH_BLOCK = 512
GROUP_BLOCK = 8
NCLUST = 4


def _up(arr, s):
    return jnp.concatenate([arr[s:], arr[:s]], axis=0)


def _down(arr, s):
    n = arr.shape[0]
    return jnp.concatenate([arr[n - s:], arr[:n - s]], axis=0)


def _hoa_body(x_ref, ch_ref, cl_ref, o_ref):
    nd = GROUP_BLOCK * 32
    nr = NCLUST * GROUP_BLOCK * 4
    xt = jnp.transpose(x_ref[...])
    r1, r2, r3 = _up(xt, 1), _up(xt, 2), _up(xt, 3)
    p1 = jnp.minimum(xt, r1)
    p2 = jnp.minimum(xt, r2)
    p3 = jnp.minimum(xt, r3)
    t1 = jnp.minimum(p1, r2)
    t2 = jnp.minimum(p1, r3)
    t3 = jnp.minimum(p2, r3)
    q = jnp.minimum(t1, r3)
    mx = jnp.maximum(jnp.maximum(xt, r1), jnp.maximum(r2, r3))
    pos = jax.lax.broadcasted_iota(jnp.int32, xt.shape, 0) % 4
    A0 = xt
    A1 = jnp.where(pos == 3, _down(t2, 3), p1)
    A2 = jnp.where(pos < 2, p2, _down(t1, 2))
    A3 = jnp.where(pos == 0, p3,
                   jnp.where(pos == 1, _down(t3, 1),
                             jnp.where(pos == 2, _down(q, 2), _down(mx, 3))))
    dn = (((0,), (0,)), ((), ()))
    for jj in range(NCLUST):
        s = jj * 4 * GROUP_BLOCK
        e = s + 4 * GROUP_BLOCK
        m = jnp.concatenate([A0[s:e], A1[s:e], A2[s:e], A3[s:e]], axis=0)
        mh = m.astype(jnp.bfloat16).astype(jnp.float32)
        ml = m - mh
        ch = ch_ref[jj]
        cl = cl_ref[jj]
        o_ref[:, jj * nd:(jj + 1) * nd] = (
            lax.dot_general(mh, ch, dn, preferred_element_type=jnp.float32)
            + lax.dot_general(ml, ch, dn, preferred_element_type=jnp.float32)
            + lax.dot_general(mh, cl, dn, preferred_element_type=jnp.float32))


def kernel(X, params):
    B, G, A = X.shape
    D = params.shape[-1]
    GB, BB = GROUP_BLOCK, BATCH_BLOCK

    X2 = X.reshape(B, G * A)

    c = params.reshape(G, 2, 2, 2, 2, D)
    for ax in (1, 2, 3, 4):
        lo = lax.slice_in_dim(c, 0, 1, axis=ax)
        hi = lax.slice_in_dim(c, 1, 2, axis=ax)
        c = jnp.concatenate([lo, hi - lo], axis=ax)
    cmob = c.reshape(G, NSUB, D)
    cmob = cmob.at[:, 0, :].set(params[:, 0, :])
    tidx = np.array([[1, 2, 4, 8],
                     [3, 6, 12, 11],
                     [5, 10, 7, 14],
                     [9, 13, 15, 0]], dtype=np.int32)
    cm = cmob.reshape(G // GB, GB, NSUB, D)
    sel = jnp.take(cm, jnp.asarray(tidx.reshape(-1)), axis=2)
    sel = sel.reshape(G // GB, GB, 4, 4, D)
    cbd = jnp.einsum('cjkid,jJ->ckjiJd', sel, jnp.eye(GB, dtype=cmob.dtype))
    cbd = cbd.reshape(G // GB, NSUB * GB, GB * D)
    cbd_h = cbd.astype(jnp.bfloat16).astype(jnp.float32)
    cbd_l = cbd - cbd_h

    grid = (G // (GB * NCLUST), B // BB)
    out2 = pl.pallas_call(
        _hoa_body,
        grid=grid,
        in_specs=[
            pl.BlockSpec((BB, NCLUST * GB * A), lambda g, b: (b, g)),
            pl.BlockSpec((NCLUST, NSUB * GB, GB * D), lambda g, b: (g, 0, 0)),
            pl.BlockSpec((NCLUST, NSUB * GB, GB * D), lambda g, b: (g, 0, 0)),
        ],
        out_specs=pl.BlockSpec((BB, NCLUST * GB * D), lambda g, b: (b, g)),
        out_shape=jax.ShapeDtypeStruct((B, G * D), jnp.float32),
    )(X2, cbd_h, cbd_l)
    return out2.reshape(B, G, D)


# speedup vs baseline: 1.9748x; 1.0037x over previous
import jax
import jax.numpy as jnp
import numpy as np
from jax import lax
from jax.experimental import pallas as pl

NSUB = 16
BATCH_BLOCK = 512
GROUP_BLOCK = 8
NCLUST = 4


def _up(arr, s):
    return jnp.concatenate([arr[s:], arr[:s]], axis=0)


def _down(arr, s):
    n = arr.shape[0]
    return jnp.concatenate([arr[n - s:], arr[:n - s]], axis=0)


def _hoa_body(x_ref, ch_ref, cl_ref, o_ref):
    nd = GROUP_BLOCK * 32
    nr = NCLUST * GROUP_BLOCK * 4
    xt = jnp.transpose(x_ref[...])
    r1, r2, r3 = _up(xt, 1), _up(xt, 2), _up(xt, 3)
    p1 = jnp.minimum(xt, r1)
    p2 = jnp.minimum(xt, r2)
    p3 = jnp.minimum(xt, r3)
    t1 = jnp.minimum(p1, r2)
    t2 = jnp.minimum(p1, r3)
    t3 = jnp.minimum(p2, r3)
    q = jnp.minimum(t1, r3)
    mx = jnp.maximum(jnp.maximum(xt, r1), jnp.maximum(r2, r3))
    pos = jax.lax.broadcasted_iota(jnp.int32, xt.shape, 0) % 4
    A0 = xt
    A1 = jnp.where(pos == 3, _down(t2, 3), p1)
    A2 = jnp.where(pos < 2, p2, _down(t1, 2))
    A3 = jnp.where(pos == 0, p3,
                   jnp.where(pos == 1, _down(t3, 1),
                             jnp.where(pos == 2, _down(q, 2), _down(mx, 3))))
    dn = (((0,), (0,)), ((), ()))
    for jj in range(NCLUST):
        s = jj * 4 * GROUP_BLOCK
        e = s + 4 * GROUP_BLOCK
        m = jnp.concatenate([A0[s:e], A1[s:e], A2[s:e], A3[s:e]], axis=0)
        mh = lax.bitcast_convert_type(
            lax.bitcast_convert_type(m, jnp.int32) & jnp.int32(-65536),
            jnp.float32)
        ml = m - mh
        ch = ch_ref[jj]
        cl = cl_ref[jj]
        o_ref[:, jj * nd:(jj + 1) * nd] = (
            lax.dot_general(mh, ch, dn, preferred_element_type=jnp.float32)
            + lax.dot_general(ml, ch, dn, preferred_element_type=jnp.float32)
            + lax.dot_general(mh, cl, dn, preferred_element_type=jnp.float32))


def kernel(X, params):
    B, G, A = X.shape
    D = params.shape[-1]
    GB, BB = GROUP_BLOCK, BATCH_BLOCK

    X2 = X.reshape(B, G * A)

    c = params.reshape(G, 2, 2, 2, 2, D)
    for ax in (1, 2, 3, 4):
        lo = lax.slice_in_dim(c, 0, 1, axis=ax)
        hi = lax.slice_in_dim(c, 1, 2, axis=ax)
        c = jnp.concatenate([lo, hi - lo], axis=ax)
    cmob = c.reshape(G, NSUB, D)
    cmob = cmob.at[:, 0, :].set(params[:, 0, :])
    tidx = np.array([[1, 2, 4, 8],
                     [3, 6, 12, 11],
                     [5, 10, 7, 14],
                     [9, 13, 15, 0]], dtype=np.int32)
    cm = cmob.reshape(G // GB, GB, NSUB, D)
    sel = jnp.take(cm, jnp.asarray(tidx.reshape(-1)), axis=2)
    sel = sel.reshape(G // GB, GB, 4, 4, D)
    cbd = jnp.einsum('cjkid,jJ->ckjiJd', sel, jnp.eye(GB, dtype=cmob.dtype))
    cbd = cbd.reshape(G // GB, NSUB * GB, GB * D)
    cbd_h = lax.bitcast_convert_type(
        lax.bitcast_convert_type(cbd, jnp.int32) & jnp.int32(-65536),
        jnp.float32)
    cbd_l = cbd - cbd_h

    grid = (G // (GB * NCLUST), B // BB)
    out2 = pl.pallas_call(
        _hoa_body,
        grid=grid,
        in_specs=[
            pl.BlockSpec((BB, NCLUST * GB * A), lambda g, b: (b, g)),
            pl.BlockSpec((NCLUST, NSUB * GB, GB * D), lambda g, b: (g, 0, 0)),
            pl.BlockSpec((NCLUST, NSUB * GB, GB * D), lambda g, b: (g, 0, 0)),
        ],
        out_specs=pl.BlockSpec((BB, NCLUST * GB * D), lambda g, b: (b, g)),
        out_shape=jax.ShapeDtypeStruct((B, G * D), jnp.float32),
    )(X2, cbd_h, cbd_l)
    return out2.reshape(B, G, D)


# speedup vs baseline: 2.1610x; 1.0943x over previous
import jax
import jax.numpy as jnp
import numpy as np
from jax import lax
from jax.experimental import pallas as pl

NSUB = 16
BATCH_BLOCK = 512
GROUP_BLOCK = 8


def _hoa_body(x0_ref, x1_ref, x2_ref, x3_ref, ch_ref, cl_ref, o_ref):
    nd = GROUP_BLOCK * 32
    a0 = jnp.transpose(x0_ref[...])
    a1 = jnp.transpose(x1_ref[...])
    a2 = jnp.transpose(x2_ref[...])
    a3 = jnp.transpose(x3_ref[...])
    m3 = jnp.minimum(a0, a1)
    m5 = jnp.minimum(a0, a2)
    m6 = jnp.minimum(a1, a2)
    m9 = jnp.minimum(a0, a3)
    m10 = jnp.minimum(a1, a3)
    m12 = jnp.minimum(a2, a3)
    m7 = jnp.minimum(m3, a2)
    m11 = jnp.minimum(m3, a3)
    m13 = jnp.minimum(m5, a3)
    m14 = jnp.minimum(m6, a3)
    m15 = jnp.minimum(m7, a3)
    m0 = jnp.maximum(jnp.maximum(a0, a1), jnp.maximum(a2, a3))
    rows = [m0, a0, a1, m3, a2, m5, m6, m7, a3, m9, m10, m11, m12, m13, m14, m15]
    dn = (((0,), (0,)), ((), ()))
    nclust = a0.shape[0] // GROUP_BLOCK
    for jj in range(nclust):
        s = jj * GROUP_BLOCK
        e = s + GROUP_BLOCK
        m = jnp.concatenate([r[s:e] for r in rows], axis=0)
        mh = lax.bitcast_convert_type(
            lax.bitcast_convert_type(m, jnp.int32) & jnp.int32(-65536),
            jnp.float32)
        ml = m - mh
        ch = ch_ref[jj]
        cl = cl_ref[jj]
        o_ref[:, jj * nd:(jj + 1) * nd] = (
            lax.dot_general(mh, ch, dn, preferred_element_type=jnp.float32)
            + lax.dot_general(ml, ch, dn, preferred_element_type=jnp.float32)
            + lax.dot_general(mh, cl, dn, preferred_element_type=jnp.float32))


def kernel(X, params):
    B, G, A = X.shape
    D = params.shape[-1]
    GB, BB = GROUP_BLOCK, BATCH_BLOCK

    xs = [X[:, :, i] for i in range(A)]

    c = params.reshape(G, 2, 2, 2, 2, D)
    for ax in (1, 2, 3, 4):
        lo = lax.slice_in_dim(c, 0, 1, axis=ax)
        hi = lax.slice_in_dim(c, 1, 2, axis=ax)
        c = jnp.concatenate([lo, hi - lo], axis=ax)
    cmob = c.reshape(G, NSUB, D)
    cmob = cmob.at[:, 0, :].set(params[:, 0, :])
    cbd = jnp.einsum('cjtd,jk->ctjkd', cmob.reshape(G // GB, GB, NSUB, D),
                     jnp.eye(GB, dtype=cmob.dtype))
    cbd = cbd.reshape(G // GB, NSUB * GB, GB * D)
    cbd_h = lax.bitcast_convert_type(
        lax.bitcast_convert_type(cbd, jnp.int32) & jnp.int32(-65536),
        jnp.float32)
    cbd_l = cbd - cbd_h

    grid = (B // BB,)
    xspec = pl.BlockSpec((BB, G), lambda b: (b, 0))
    cspec = pl.BlockSpec((G // GB, NSUB * GB, GB * D), lambda b: (0, 0, 0))
    out2 = pl.pallas_call(
        _hoa_body,
        grid=grid,
        in_specs=[xspec, xspec, xspec, xspec, cspec, cspec],
        out_specs=pl.BlockSpec((BB, G * D), lambda b: (b, 0)),
        out_shape=jax.ShapeDtypeStruct((B, G * D), jnp.float32),
    )(*xs, cbd_h, cbd_l)
    return out2.reshape(B, G, D)
